# Initial kernel scaffold; baseline (speedup 1.0000x reference)
import functools

import jax
import jax.numpy as jnp
from jax import lax
from jax.experimental import pallas as pl
from jax.experimental.pallas import tpu as pltpu
from jax.experimental.pallas import tpu_sc as plsc

N = 10000
E = 320000
D = 128
H = 128
C = 16
G = 64

NC = 2
NS = 16
NW = NC * NS
EPW = E // NW
K = 80
NB = EPW // K
NP = 10240
ZROW = N
ZB = 64
RPT = NP // NS

_mesh = plsc.VectorSubcoreMesh(core_axis_name="c", subcore_axis_name="s")


def _zero_vmem(ref, nrow, ncol):
    def body(i, carry):
        for c in range(ncol // 16):
            ref[i, pl.ds(c * 16, 16)] = jnp.zeros((16,), jnp.float32)
        return carry
    lax.fori_loop(0, nrow, body, 0)


@functools.partial(
    pl.kernel,
    out_type=(
        jax.ShapeDtypeStruct((NC, NP, D), jnp.float32),
        jax.ShapeDtypeStruct((NC, NP, 16), jnp.float32),
    ),
    mesh=_mesh,
    scratch_types=[
        pltpu.VMEM((NB, K), jnp.int32),
        pltpu.VMEM((NB, K), jnp.int32),
        pltpu.VMEM((NB, K), jnp.float32),
        pltpu.VMEM((K,), jnp.int32),
        pltpu.VMEM((K, D), jnp.float32),
        pltpu.VMEM((K, 16), jnp.float32),
        pltpu.VMEM((ZB, D), jnp.float32),
        pltpu.VMEM((ZB, 16), jnp.float32),
        pltpu.VMEM_SHARED((NP, D), jnp.float32),
        pltpu.VMEM_SHARED((NP, 16), jnp.float32),
        pltpu.SemaphoreType.DMA,
    ],
)
def _sc_pass_a(x_hbm, src_hbm, dst_hbm, msk_hbm, out_hbm, deg_hbm,
               src_v, dst_v, msk_v, idx_v, rows_v, ones_v, zf_v, zd_v,
               acc_sh, deg_sh, sem):
    cid = lax.axis_index("c")
    sid = lax.axis_index("s")
    wid = cid * NS + sid

    _zero_vmem(zf_v, ZB, D)
    _zero_vmem(zd_v, ZB, 16)

    def ones_body(i, carry):
        ones_v[i, :] = jnp.where(lax.iota(jnp.int32, 16) == 0, 1.0, 0.0)
        return carry
    lax.fori_loop(0, K, ones_body, 0)

    base = sid * RPT
    for t in range(RPT // ZB):
        pltpu.sync_copy(zf_v, acc_sh.at[pl.ds(base + t * ZB, ZB)])
        pltpu.sync_copy(zd_v, deg_sh.at[pl.ds(base + t * ZB, ZB)])
    plsc.subcore_barrier()

    pltpu.sync_copy(src_hbm.at[wid], src_v)
    pltpu.sync_copy(dst_hbm.at[wid], dst_v)
    pltpu.sync_copy(msk_hbm.at[wid], msk_v)

    def batch(j, carry):
        for i in range(K // 16):
            m = msk_v[j, pl.ds(i * 16, 16)]
            s = src_v[j, pl.ds(i * 16, 16)]
            idx_v[pl.ds(i * 16, 16)] = jnp.where(m > 0.5, s, ZROW)
        pltpu.async_copy(x_hbm.at[idx_v], rows_v, sem).wait()
        pltpu.sync_copy(rows_v, acc_sh.at[dst_v.at[j]], add=True)
        pltpu.sync_copy(ones_v, deg_sh.at[dst_v.at[j]], add=True)
        return carry
    lax.fori_loop(0, NB, batch, 0)

    plsc.subcore_barrier()
    pltpu.sync_copy(acc_sh.at[pl.ds(base, RPT)], out_hbm.at[cid, pl.ds(base, RPT)])
    pltpu.sync_copy(deg_sh.at[pl.ds(base, RPT)], deg_hbm.at[cid, pl.ds(base, RPT)])


@functools.partial(
    pl.kernel,
    out_type=jax.ShapeDtypeStruct((NC, NP, D), jnp.float32),
    mesh=_mesh,
    scratch_types=[
        pltpu.VMEM((NB, K), jnp.int32),
        pltpu.VMEM((NB, K), jnp.int32),
        pltpu.VMEM((K, D), jnp.float32),
        pltpu.VMEM((ZB, D), jnp.float32),
        pltpu.VMEM_SHARED((NP, D), jnp.float32),
        pltpu.SemaphoreType.DMA,
    ],
)
def _sc_pass_b(y_hbm, src_hbm, dst_hbm, out_hbm,
               src_v, dst_v, rows_v, zf_v, acc_sh, sem):
    cid = lax.axis_index("c")
    sid = lax.axis_index("s")
    wid = cid * NS + sid

    _zero_vmem(zf_v, ZB, D)
    base = sid * RPT
    for t in range(RPT // ZB):
        pltpu.sync_copy(zf_v, acc_sh.at[pl.ds(base + t * ZB, ZB)])
    plsc.subcore_barrier()

    pltpu.sync_copy(src_hbm.at[wid], src_v)
    pltpu.sync_copy(dst_hbm.at[wid], dst_v)

    def batch(j, carry):
        pltpu.async_copy(y_hbm.at[src_v.at[j]], rows_v, sem).wait()
        pltpu.sync_copy(rows_v, acc_sh.at[dst_v.at[j]], add=True)
        return carry
    lax.fori_loop(0, NB, batch, 0)

    plsc.subcore_barrier()
    pltpu.sync_copy(acc_sh.at[pl.ds(base, RPT)], out_hbm.at[cid, pl.ds(base, RPT)])


BR = 128
NBLK = NP // BR


def _dis_from_deg(d0_blk, d1_blk):
    deg = d0_blk[:, :1] + d1_blk[:, :1] + 1.0
    return lax.rsqrt(jnp.maximum(deg, 1e-12))


def _tc1_body(x_ref, a0_ref, a1_ref, sm_ref, d0_ref, d1_ref, w_ref,
              y_ref, dis_ref):
    sm = sm_ref[...]
    x2 = a0_ref[...] + a1_ref[...]
    xm = sm * x2 + (1.0 - sm) * x_ref[...]
    dis = _dis_from_deg(d0_ref[...], d1_ref[...])
    y_ref[...] = dis * jnp.dot(xm, w_ref[...], preferred_element_type=jnp.float32)
    dis_ref[...] = dis


def _tc_mid_body(a0_ref, a1_ref, y_ref, dis_ref, b_ref, w_ref, out_ref):
    dis = dis_ref[...]
    h = dis * (a0_ref[...] + a1_ref[...] + y_ref[...]) + b_ref[...]
    h = jnp.maximum(h, 0.0)
    out_ref[...] = dis * jnp.dot(h, w_ref[...], preferred_element_type=jnp.float32)


def _tc4_body(a0_ref, a1_ref, y_ref, dis_ref, b_ref, bat_ref, wl_ref, bl_ref,
              out_ref, sums, cnt):
    i = pl.program_id(0)

    @pl.when(i == 0)
    def _init():
        sums[...] = jnp.zeros_like(sums)
        cnt[...] = jnp.zeros_like(cnt)

    dis = dis_ref[...]
    h = dis * (a0_ref[...] + a1_ref[...] + y_ref[...]) + b_ref[...]
    gids = lax.broadcasted_iota(jnp.int32, (BR, G), 1)
    oh = (bat_ref[...] == gids).astype(jnp.float32)
    sums[...] += lax.dot_general(oh, h, (((0,), (0,)), ((), ())),
                                 preferred_element_type=jnp.float32)
    cnt[...] += lax.dot_general(oh, jnp.ones((BR, 1), jnp.float32),
                                (((0,), (0,)), ((), ())),
                                preferred_element_type=jnp.float32)

    @pl.when(i == NBLK - 1)
    def _fin():
        pooled = sums[...] / jnp.maximum(cnt[...], 1.0)
        out_ref[...] = jnp.dot(pooled, wl_ref[...],
                               preferred_element_type=jnp.float32) + bl_ref[...]


def _row_spec(cols):
    return pl.BlockSpec((BR, cols), lambda i: (i, 0))


def _full_spec(rows, cols):
    return pl.BlockSpec((rows, cols), lambda i: (0, 0))


def kernel(x, edge_index, supernode_mask, edge_mask, batch,
           W1, b1, W2, b2, W3, b3, Wl, bl):
    f32 = jnp.float32
    src = edge_index[0].reshape(NW, NB, K)
    dst = edge_index[1].reshape(NW, NB, K)
    msk = edge_mask.reshape(NW, NB, K).astype(f32)

    pad = NP - N
    x_pad = jnp.concatenate([x, jnp.zeros((pad, D), f32)], axis=0)
    sm_col = jnp.concatenate(
        [supernode_mask.astype(f32), jnp.zeros((pad,), f32)])[:, None]
    bat_col = jnp.concatenate(
        [batch, jnp.full((pad,), G, jnp.int32)])[:, None]

    parts_a, deg_parts = _sc_pass_a(x_pad, src, dst, msk)

    y1, dis = pl.pallas_call(
        _tc1_body,
        grid=(NBLK,),
        in_specs=[_row_spec(D), _row_spec(D), _row_spec(D), _row_spec(1),
                  _row_spec(16), _row_spec(16), _full_spec(D, H)],
        out_specs=[_row_spec(H), _row_spec(1)],
        out_shape=[jax.ShapeDtypeStruct((NP, H), f32),
                   jax.ShapeDtypeStruct((NP, 1), f32)],
    )(x_pad, parts_a[0], parts_a[1], sm_col, deg_parts[0], deg_parts[1], W1)

    def mid_layer(y_in, b_in, w_next):
        parts = _sc_pass_b(y_in, src, dst)
        return pl.pallas_call(
            _tc_mid_body,
            grid=(NBLK,),
            in_specs=[_row_spec(H), _row_spec(H), _row_spec(H), _row_spec(1),
                      _full_spec(1, H), _full_spec(H, H)],
            out_specs=_row_spec(H),
            out_shape=jax.ShapeDtypeStruct((NP, H), f32),
        )(parts[0], parts[1], y_in, dis, b_in[None, :], w_next)

    y2 = mid_layer(y1, b1, W2)
    y3 = mid_layer(y2, b2, W3)

    parts3 = _sc_pass_b(y3, src, dst)
    out = pl.pallas_call(
        _tc4_body,
        grid=(NBLK,),
        in_specs=[_row_spec(H), _row_spec(H), _row_spec(H), _row_spec(1),
                  _full_spec(1, H), _row_spec(1), _full_spec(H, C),
                  _full_spec(1, C)],
        out_specs=_full_spec(G, C),
        out_shape=jax.ShapeDtypeStruct((G, C), f32),
        scratch_shapes=[pltpu.VMEM((G, H), f32), pltpu.VMEM((G, 1), f32)],
    )(parts3[0], parts3[1], y3, dis, b3[None, :], bat_col, Wl, bl[None, :])
    return out


# speedup vs baseline: 1.7459x; 1.7459x over previous
import functools

import jax
import jax.numpy as jnp
from jax import lax
from jax.experimental import pallas as pl
from jax.experimental.pallas import tpu as pltpu
from jax.experimental.pallas import tpu_sc as plsc

N = 10000
E = 320000
D = 128
H = 128
C = 16
G = 64

NC = 2
NS = 16
NW = NC * NS
EPW = E // NW
K = 80
NB = EPW // K
NP = 10240
ZROW = N
ZB = 32
RPT = NP // NS
CB = 25
NCH = NB // CB

_mesh = plsc.VectorSubcoreMesh(core_axis_name="c", subcore_axis_name="s")


def _zero_vmem(ref, nrow, ncol):
    def body(i, carry):
        for c in range(ncol // 16):
            ref[i, pl.ds(c * 16, 16)] = jnp.zeros((16,), jnp.float32)
        return carry
    lax.fori_loop(0, nrow, body, 0)


@functools.partial(
    pl.kernel,
    out_type=jax.ShapeDtypeStruct((NC, NP, D), jnp.float32),
    mesh=_mesh,
    scratch_types=[
        pltpu.VMEM((CB, K), jnp.int32),
        pltpu.VMEM((CB, K), jnp.int32),
        pltpu.VMEM((CB, K), jnp.float32),
        pltpu.VMEM((K,), jnp.int32),
        pltpu.VMEM((K, D), jnp.float32),
        pltpu.VMEM((ZB, D), jnp.float32),
        pltpu.VMEM_SHARED((NP, D), jnp.float32),
        pltpu.SemaphoreType.DMA,
    ],
)
def _sc_edge_pass(x_hbm, src_hbm, dst_hbm, msk_hbm, out_hbm,
                  src_v, dst_v, msk_v, idx_v, rows_v, zf_v, acc_sh, sem):
    cid = lax.axis_index("c")
    sid = lax.axis_index("s")
    wid = cid * NS + sid

    _zero_vmem(zf_v, ZB, D)

    base = sid * RPT
    for t in range(RPT // ZB):
        pltpu.sync_copy(zf_v, acc_sh.at[pl.ds(base + t * ZB, ZB)])
    plsc.subcore_barrier()

    def batch(j, carry):
        for i in range(K // 16):
            m = msk_v[j, pl.ds(i * 16, 16)]
            s = src_v[j, pl.ds(i * 16, 16)]
            idx_v[pl.ds(i * 16, 16)] = jnp.where(m > 0.5, s, ZROW)
        pltpu.async_copy(x_hbm.at[idx_v], rows_v, sem).wait()
        pltpu.sync_copy(rows_v, acc_sh.at[dst_v.at[j]], add=True)
        return carry

    for c in range(NCH):
        pltpu.sync_copy(src_hbm.at[wid, c], src_v)
        pltpu.sync_copy(dst_hbm.at[wid, c], dst_v)
        pltpu.sync_copy(msk_hbm.at[wid, c], msk_v)
        lax.fori_loop(0, CB, batch, 0)

    plsc.subcore_barrier()
    pltpu.sync_copy(acc_sh.at[pl.ds(base, RPT)], out_hbm.at[cid, pl.ds(base, RPT)])


BR = 128
NBLK = NP // BR


def _dis_from_deg(d0_blk, d1_blk):
    deg = d0_blk[:, :1] + d1_blk[:, :1] + 1.0
    return lax.rsqrt(jnp.maximum(deg, 1e-12))


def _tc1_body(x_ref, a0_ref, a1_ref, sm_ref, d0_ref, d1_ref, w_ref,
              y_ref, dis_ref):
    sm = sm_ref[...]
    x2 = a0_ref[...] + a1_ref[...]
    xm = sm * x2 + (1.0 - sm) * x_ref[...]
    dis = _dis_from_deg(d0_ref[...], d1_ref[...])
    y_ref[...] = dis * jnp.dot(xm, w_ref[...], preferred_element_type=jnp.float32)
    dis_ref[...] = dis


def _tc_mid_body(a0_ref, a1_ref, y_ref, dis_ref, b_ref, w_ref, out_ref):
    dis = dis_ref[...]
    h = dis * (a0_ref[...] + a1_ref[...] + y_ref[...]) + b_ref[...]
    h = jnp.maximum(h, 0.0)
    out_ref[...] = dis * jnp.dot(h, w_ref[...], preferred_element_type=jnp.float32)


def _tc4_body(a0_ref, a1_ref, y_ref, dis_ref, b_ref, bat_ref, wl_ref, bl_ref,
              out_ref, sums, cnt):
    i = pl.program_id(0)

    @pl.when(i == 0)
    def _init():
        sums[...] = jnp.zeros_like(sums)
        cnt[...] = jnp.zeros_like(cnt)

    dis = dis_ref[...]
    h = dis * (a0_ref[...] + a1_ref[...] + y_ref[...]) + b_ref[...]
    gids = lax.broadcasted_iota(jnp.int32, (BR, G), 1)
    oh = (bat_ref[...] == gids).astype(jnp.float32)
    sums[...] += lax.dot_general(oh, h, (((0,), (0,)), ((), ())),
                                 preferred_element_type=jnp.float32)
    cnt[...] += lax.dot_general(oh, jnp.ones((BR, 1), jnp.float32),
                                (((0,), (0,)), ((), ())),
                                preferred_element_type=jnp.float32)

    @pl.when(i == NBLK - 1)
    def _fin():
        pooled = sums[...] / jnp.maximum(cnt[...], 1.0)
        out_ref[...] = jnp.dot(pooled, wl_ref[...],
                               preferred_element_type=jnp.float32) + bl_ref[...]


def _row_spec(cols):
    return pl.BlockSpec((BR, cols), lambda i: (i, 0))


def _full_spec(rows, cols):
    return pl.BlockSpec((rows, cols), lambda i: (0, 0))


def kernel(x, edge_index, supernode_mask, edge_mask, batch,
           W1, b1, W2, b2, W3, b3, Wl, bl):
    f32 = jnp.float32
    src = edge_index[0].reshape(NW, NCH, CB, K)
    dst = edge_index[1].reshape(NW, NCH, CB, K)
    msk = edge_mask.reshape(NW, NCH, CB, K).astype(f32)

    pad = NP - N
    x_pad = jnp.concatenate([x, jnp.zeros((pad, D), f32)], axis=0)
    sm_col = jnp.concatenate(
        [supernode_mask.astype(f32), jnp.zeros((pad,), f32)])[:, None]
    bat_col = jnp.concatenate(
        [batch, jnp.full((pad,), G, jnp.int32)])[:, None]

    ones_msk = jnp.ones((NW, NCH, CB, K), f32)

    e0_table = jnp.zeros((NP, D), f32).at[:, 0].set(1.0)
    deg_parts = _sc_edge_pass(e0_table, dst, dst, ones_msk)
    parts_a = _sc_edge_pass(x_pad, src, dst, msk)

    y1, dis = pl.pallas_call(
        _tc1_body,
        grid=(NBLK,),
        in_specs=[_row_spec(D), _row_spec(D), _row_spec(D), _row_spec(1),
                  _row_spec(D), _row_spec(D), _full_spec(D, H)],
        out_specs=[_row_spec(H), _row_spec(1)],
        out_shape=[jax.ShapeDtypeStruct((NP, H), f32),
                   jax.ShapeDtypeStruct((NP, 1), f32)],
    )(x_pad, parts_a[0], parts_a[1], sm_col, deg_parts[0], deg_parts[1], W1)

    def mid_layer(y_in, b_in, w_next):
        parts = _sc_edge_pass(y_in, src, dst, ones_msk)
        return pl.pallas_call(
            _tc_mid_body,
            grid=(NBLK,),
            in_specs=[_row_spec(H), _row_spec(H), _row_spec(H), _row_spec(1),
                      _full_spec(1, H), _full_spec(H, H)],
            out_specs=_row_spec(H),
            out_shape=jax.ShapeDtypeStruct((NP, H), f32),
        )(parts[0], parts[1], y_in, dis, b_in[None, :], w_next)

    y2 = mid_layer(y1, b1, W2)
    y3 = mid_layer(y2, b2, W3)

    parts3 = _sc_edge_pass(y3, src, dst, ones_msk)
    out = pl.pallas_call(
        _tc4_body,
        grid=(NBLK,),
        in_specs=[_row_spec(H), _row_spec(H), _row_spec(H), _row_spec(1),
                  _full_spec(1, H), _row_spec(1), _full_spec(H, C),
                  _full_spec(1, C)],
        out_specs=_full_spec(G, C),
        out_shape=jax.ShapeDtypeStruct((G, C), f32),
        scratch_shapes=[pltpu.VMEM((G, H), f32), pltpu.VMEM((G, 1), f32)],
    )(parts3[0], parts3[1], y3, dis, b3[None, :], bat_col, Wl, bl[None, :])
    return out


# speedup vs baseline: 9.1848x; 5.2607x over previous
import functools

import jax
import jax.numpy as jnp
from jax import lax
from jax.experimental import pallas as pl
from jax.experimental.pallas import tpu as pltpu
from jax.experimental.pallas import tpu_sc as plsc

N = 10000
E = 320000
D = 128
H = 128
C = 16
G = 64

NC = 2
NS = 16
NW = NC * NS
EPW = E // NW
K = 80
NB = EPW // K
NP = 10240
ZROW = N
ZB = 32
RPT = NP // NS
CB = 25
NCH = NB // CB

_mesh = plsc.VectorSubcoreMesh(core_axis_name="c", subcore_axis_name="s")


def _zero_vmem(ref, nrow, ncol):
    def body(i, carry):
        for c in range(ncol // 16):
            ref[i, pl.ds(c * 16, 16)] = jnp.zeros((16,), jnp.float32)
        return carry
    lax.fori_loop(0, nrow, body, 0)


@functools.partial(
    pl.kernel,
    out_type=jax.ShapeDtypeStruct((NC, NP, D), jnp.float32),
    mesh=_mesh,
    scratch_types=[
        pltpu.VMEM((CB, K), jnp.int32),
        pltpu.VMEM((CB, K), jnp.int32),
        pltpu.VMEM((CB, K), jnp.float32),
        pltpu.VMEM((K,), jnp.int32),
        pltpu.VMEM((K,), jnp.int32),
        pltpu.VMEM((K, D), jnp.float32),
        pltpu.VMEM((ZB, D), jnp.float32),
        pltpu.VMEM_SHARED((NP, D), jnp.float32),
        pltpu.SemaphoreType.DMA,
    ],
)
def _sc_edge_pass(x_hbm, src_hbm, dst_hbm, msk_hbm, pad_hbm, out_hbm,
                  src_v, dst_v, msk_v, idx_v, pad_v, rows_v, zf_v, acc_sh,
                  sem):
    cid = lax.axis_index("c")
    sid = lax.axis_index("s")
    wid = cid * NS + sid

    _zero_vmem(zf_v, ZB, D)

    base = sid * RPT
    for t in range(RPT // ZB):
        pltpu.sync_copy(zf_v, acc_sh.at[pl.ds(base + t * ZB, ZB)])
    plsc.subcore_barrier()

    pltpu.sync_copy(pad_hbm, pad_v)

    def batch(j, carry):
        for i in range(K // 16):
            m = msk_v[j, pl.ds(i * 16, 16)]
            s = src_v[j, pl.ds(i * 16, 16)]
            z = pad_v[pl.ds(i * 16, 16)]
            idx_v[pl.ds(i * 16, 16)] = jnp.where(m > 0.5, s, z)
        pltpu.async_copy(x_hbm.at[idx_v], rows_v, sem).wait()
        pltpu.sync_copy(rows_v, acc_sh.at[dst_v.at[j]], add=True)
        return carry

    for c in range(NCH):
        pltpu.sync_copy(src_hbm.at[wid, c], src_v)
        pltpu.sync_copy(dst_hbm.at[wid, c], dst_v)
        pltpu.sync_copy(msk_hbm.at[wid, c], msk_v)
        lax.fori_loop(0, CB, batch, 0)

    plsc.subcore_barrier()
    pltpu.sync_copy(acc_sh.at[pl.ds(base, RPT)], out_hbm.at[cid, pl.ds(base, RPT)])


BR = 128
NBLK = NP // BR


def _dis_from_deg(d0_blk, d1_blk):
    deg = d0_blk[:, :1] + d1_blk[:, :1] + 1.0
    return lax.rsqrt(jnp.maximum(deg, 1e-12))


def _tc1_body(x_ref, a0_ref, a1_ref, sm_ref, d0_ref, d1_ref, w_ref,
              y_ref, dis_ref):
    sm = sm_ref[...]
    x2 = a0_ref[...] + a1_ref[...]
    xm = sm * x2 + (1.0 - sm) * x_ref[...]
    dis = _dis_from_deg(d0_ref[...], d1_ref[...])
    y_ref[...] = dis * jnp.dot(xm, w_ref[...], preferred_element_type=jnp.float32)
    dis_ref[...] = dis


def _tc_mid_body(a0_ref, a1_ref, y_ref, dis_ref, b_ref, w_ref, out_ref):
    dis = dis_ref[...]
    h = dis * (a0_ref[...] + a1_ref[...] + y_ref[...]) + b_ref[...]
    h = jnp.maximum(h, 0.0)
    out_ref[...] = dis * jnp.dot(h, w_ref[...], preferred_element_type=jnp.float32)


def _tc4_body(a0_ref, a1_ref, y_ref, dis_ref, b_ref, bat_ref, wl_ref, bl_ref,
              out_ref, sums, cnt):
    i = pl.program_id(0)

    @pl.when(i == 0)
    def _init():
        sums[...] = jnp.zeros_like(sums)
        cnt[...] = jnp.zeros_like(cnt)

    dis = dis_ref[...]
    h = dis * (a0_ref[...] + a1_ref[...] + y_ref[...]) + b_ref[...]
    gids = lax.broadcasted_iota(jnp.int32, (BR, G), 1)
    oh = (bat_ref[...] == gids).astype(jnp.float32)
    sums[...] += lax.dot_general(oh, h, (((0,), (0,)), ((), ())),
                                 preferred_element_type=jnp.float32)
    cnt[...] += lax.dot_general(oh, jnp.ones((BR, 1), jnp.float32),
                                (((0,), (0,)), ((), ())),
                                preferred_element_type=jnp.float32)

    @pl.when(i == NBLK - 1)
    def _fin():
        pooled = sums[...] / jnp.maximum(cnt[...], 1.0)
        out_ref[...] = jnp.dot(pooled, wl_ref[...],
                               preferred_element_type=jnp.float32) + bl_ref[...]


def _row_spec(cols):
    return pl.BlockSpec((BR, cols), lambda i: (i, 0))


def _full_spec(rows, cols):
    return pl.BlockSpec((rows, cols), lambda i: (0, 0))


def kernel(x, edge_index, supernode_mask, edge_mask, batch,
           W1, b1, W2, b2, W3, b3, Wl, bl):
    f32 = jnp.float32
    src = edge_index[0].reshape(NW, NCH, CB, K)
    dst = edge_index[1].reshape(NW, NCH, CB, K)
    msk = edge_mask.reshape(NW, NCH, CB, K).astype(f32)

    pad = NP - N
    x_pad = jnp.concatenate([x, jnp.zeros((pad, D), f32)], axis=0)
    sm_col = jnp.concatenate(
        [supernode_mask.astype(f32), jnp.zeros((pad,), f32)])[:, None]
    bat_col = jnp.concatenate(
        [batch, jnp.full((pad,), G, jnp.int32)])[:, None]

    ones_msk = jnp.ones((NW, NCH, CB, K), f32)

    pad_idx = (N + jnp.arange(K, dtype=jnp.int32) % (NP - N))
    e0_table = jnp.zeros((NP, D), f32).at[:, 0].set(1.0)
    deg_parts = _sc_edge_pass(e0_table, dst, dst, ones_msk, pad_idx)
    parts_a = _sc_edge_pass(x_pad, src, dst, msk, pad_idx)

    y1, dis = pl.pallas_call(
        _tc1_body,
        grid=(NBLK,),
        in_specs=[_row_spec(D), _row_spec(D), _row_spec(D), _row_spec(1),
                  _row_spec(D), _row_spec(D), _full_spec(D, H)],
        out_specs=[_row_spec(H), _row_spec(1)],
        out_shape=[jax.ShapeDtypeStruct((NP, H), f32),
                   jax.ShapeDtypeStruct((NP, 1), f32)],
    )(x_pad, parts_a[0], parts_a[1], sm_col, deg_parts[0], deg_parts[1], W1)

    def mid_layer(y_in, b_in, w_next):
        parts = _sc_edge_pass(y_in, src, dst, ones_msk, pad_idx)
        return pl.pallas_call(
            _tc_mid_body,
            grid=(NBLK,),
            in_specs=[_row_spec(H), _row_spec(H), _row_spec(H), _row_spec(1),
                      _full_spec(1, H), _full_spec(H, H)],
            out_specs=_row_spec(H),
            out_shape=jax.ShapeDtypeStruct((NP, H), f32),
        )(parts[0], parts[1], y_in, dis, b_in[None, :], w_next)

    y2 = mid_layer(y1, b1, W2)
    y3 = mid_layer(y2, b2, W3)

    parts3 = _sc_edge_pass(y3, src, dst, ones_msk, pad_idx)
    out = pl.pallas_call(
        _tc4_body,
        grid=(NBLK,),
        in_specs=[_row_spec(H), _row_spec(H), _row_spec(H), _row_spec(1),
                  _full_spec(1, H), _row_spec(1), _full_spec(H, C),
                  _full_spec(1, C)],
        out_specs=_full_spec(G, C),
        out_shape=jax.ShapeDtypeStruct((G, C), f32),
        scratch_shapes=[pltpu.VMEM((G, H), f32), pltpu.VMEM((G, 1), f32)],
    )(parts3[0], parts3[1], y3, dis, b3[None, :], bat_col, Wl, bl[None, :])
    return out


# speedup vs baseline: 12.5103x; 1.3621x over previous
import functools

import jax
import jax.numpy as jnp
from jax import lax
from jax.experimental import pallas as pl
from jax.experimental.pallas import tpu as pltpu
from jax.experimental.pallas import tpu_sc as plsc

N = 10000
E = 320000
D = 128
H = 128
C = 16
G = 64

NC = 2
NS = 16
NW = NC * NS
EPW = E // NW
K = 80
NB = EPW // K
NP = 10240
ZROW = N
ZB = 32
RPT = NP // NS
CB = 25
NCH = NB // CB

_mesh = plsc.VectorSubcoreMesh(core_axis_name="c", subcore_axis_name="s")


def _zero_vmem(ref, nrow, ncol):
    def body(i, carry):
        for c in range(ncol // 16):
            ref[i, pl.ds(c * 16, 16)] = jnp.zeros((16,), jnp.float32)
        return carry
    lax.fori_loop(0, nrow, body, 0)


@functools.partial(
    pl.kernel,
    out_type=jax.ShapeDtypeStruct((NC, NP, D), jnp.float32),
    mesh=_mesh,
    scratch_types=[
        pltpu.VMEM((CB, K), jnp.int32),
        pltpu.VMEM((CB, K), jnp.int32),
        pltpu.VMEM((CB, K), jnp.float32),
        pltpu.VMEM((K,), jnp.int32),
        pltpu.VMEM((K,), jnp.int32),
        pltpu.VMEM((K,), jnp.int32),
        pltpu.VMEM((K, D), jnp.float32),
        pltpu.VMEM((K, D), jnp.float32),
        pltpu.VMEM((ZB, D), jnp.float32),
        pltpu.VMEM_SHARED((NP, D), jnp.float32),
        pltpu.SemaphoreType.DMA,
        pltpu.SemaphoreType.DMA,
    ],
)
def _sc_edge_pass(x_hbm, src_hbm, dst_hbm, msk_hbm, pad_hbm, out_hbm,
                  src_v, dst_v, msk_v, idx0_v, idx1_v, pad_v, rows0_v,
                  rows1_v, zf_v, acc_sh, sem0, sem1):
    cid = lax.axis_index("c")
    sid = lax.axis_index("s")
    wid = cid * NS + sid

    _zero_vmem(zf_v, ZB, D)

    base = sid * RPT
    for t in range(RPT // ZB):
        pltpu.sync_copy(zf_v, acc_sh.at[pl.ds(base + t * ZB, ZB)])
    plsc.subcore_barrier()

    pltpu.sync_copy(pad_hbm, pad_v)

    def mk_idx(jj, idx_ref):
        for i in range(K // 16):
            m = msk_v[jj, pl.ds(i * 16, 16)]
            s = src_v[jj, pl.ds(i * 16, 16)]
            z = pad_v[pl.ds(i * 16, 16)]
            idx_ref[pl.ds(i * 16, 16)] = jnp.where(m > 0.5, s, z)

    for c in range(NCH):
        pltpu.sync_copy(src_hbm.at[wid, c], src_v)
        pltpu.sync_copy(dst_hbm.at[wid, c], dst_v)
        pltpu.sync_copy(msk_hbm.at[wid, c], msk_v)

        mk_idx(0, idx0_v)
        pltpu.async_copy(x_hbm.at[idx0_v], rows0_v, sem0)

        def pair(g, carry):
            j = 2 * g
            mk_idx(j + 1, idx1_v)
            pltpu.async_copy(x_hbm.at[idx1_v], rows1_v, sem1)
            pltpu.make_async_copy(x_hbm.at[idx0_v], rows0_v, sem0).wait()
            pltpu.sync_copy(rows0_v, acc_sh.at[dst_v.at[j]], add=True)

            @pl.when(j + 2 < CB)
            def _next():
                mk_idx(j + 2, idx0_v)
                pltpu.async_copy(x_hbm.at[idx0_v], rows0_v, sem0)

            pltpu.make_async_copy(x_hbm.at[idx1_v], rows1_v, sem1).wait()
            pltpu.sync_copy(rows1_v, acc_sh.at[dst_v.at[j + 1]], add=True)
            return carry

        lax.fori_loop(0, CB // 2, pair, 0)
        pltpu.make_async_copy(x_hbm.at[idx0_v], rows0_v, sem0).wait()
        pltpu.sync_copy(rows0_v, acc_sh.at[dst_v.at[CB - 1]], add=True)

    plsc.subcore_barrier()
    pltpu.sync_copy(acc_sh.at[pl.ds(base, RPT)], out_hbm.at[cid, pl.ds(base, RPT)])


BR = 128
NBLK = NP // BR


def _dis_from_deg(d0_blk, d1_blk):
    deg = d0_blk[:, :1] + d1_blk[:, :1] + 1.0
    return lax.rsqrt(jnp.maximum(deg, 1e-12))


def _tc1_body(x_ref, a0_ref, a1_ref, sm_ref, d0_ref, d1_ref, w_ref,
              y_ref, dis_ref):
    sm = sm_ref[...]
    x2 = a0_ref[...] + a1_ref[...]
    xm = sm * x2 + (1.0 - sm) * x_ref[...]
    dis = _dis_from_deg(d0_ref[...], d1_ref[...])
    y_ref[...] = dis * jnp.dot(xm, w_ref[...], preferred_element_type=jnp.float32)
    dis_ref[...] = dis


def _tc_mid_body(a0_ref, a1_ref, y_ref, dis_ref, b_ref, w_ref, out_ref):
    dis = dis_ref[...]
    h = dis * (a0_ref[...] + a1_ref[...] + y_ref[...]) + b_ref[...]
    h = jnp.maximum(h, 0.0)
    out_ref[...] = dis * jnp.dot(h, w_ref[...], preferred_element_type=jnp.float32)


def _tc4_body(a0_ref, a1_ref, y_ref, dis_ref, b_ref, bat_ref, wl_ref, bl_ref,
              out_ref, sums, cnt):
    i = pl.program_id(0)

    @pl.when(i == 0)
    def _init():
        sums[...] = jnp.zeros_like(sums)
        cnt[...] = jnp.zeros_like(cnt)

    dis = dis_ref[...]
    h = dis * (a0_ref[...] + a1_ref[...] + y_ref[...]) + b_ref[...]
    gids = lax.broadcasted_iota(jnp.int32, (BR, G), 1)
    oh = (bat_ref[...] == gids).astype(jnp.float32)
    sums[...] += lax.dot_general(oh, h, (((0,), (0,)), ((), ())),
                                 preferred_element_type=jnp.float32)
    cnt[...] += lax.dot_general(oh, jnp.ones((BR, 1), jnp.float32),
                                (((0,), (0,)), ((), ())),
                                preferred_element_type=jnp.float32)

    @pl.when(i == NBLK - 1)
    def _fin():
        pooled = sums[...] / jnp.maximum(cnt[...], 1.0)
        out_ref[...] = jnp.dot(pooled, wl_ref[...],
                               preferred_element_type=jnp.float32) + bl_ref[...]


def _row_spec(cols):
    return pl.BlockSpec((BR, cols), lambda i: (i, 0))


def _full_spec(rows, cols):
    return pl.BlockSpec((rows, cols), lambda i: (0, 0))


def kernel(x, edge_index, supernode_mask, edge_mask, batch,
           W1, b1, W2, b2, W3, b3, Wl, bl):
    f32 = jnp.float32
    src = edge_index[0].reshape(NW, NCH, CB, K)
    dst = edge_index[1].reshape(NW, NCH, CB, K)
    msk = edge_mask.reshape(NW, NCH, CB, K).astype(f32)

    pad = NP - N
    x_pad = jnp.concatenate([x, jnp.zeros((pad, D), f32)], axis=0)
    sm_col = jnp.concatenate(
        [supernode_mask.astype(f32), jnp.zeros((pad,), f32)])[:, None]
    bat_col = jnp.concatenate(
        [batch, jnp.full((pad,), G, jnp.int32)])[:, None]

    ones_msk = jnp.ones((NW, NCH, CB, K), f32)

    pad_idx = (N + jnp.arange(K, dtype=jnp.int32) % (NP - N))
    e0_table = jnp.zeros((NP, D), f32).at[:, 0].set(1.0)
    deg_parts = _sc_edge_pass(e0_table, dst, dst, ones_msk, pad_idx)
    parts_a = _sc_edge_pass(x_pad, src, dst, msk, pad_idx)

    y1, dis = pl.pallas_call(
        _tc1_body,
        grid=(NBLK,),
        in_specs=[_row_spec(D), _row_spec(D), _row_spec(D), _row_spec(1),
                  _row_spec(D), _row_spec(D), _full_spec(D, H)],
        out_specs=[_row_spec(H), _row_spec(1)],
        out_shape=[jax.ShapeDtypeStruct((NP, H), f32),
                   jax.ShapeDtypeStruct((NP, 1), f32)],
    )(x_pad, parts_a[0], parts_a[1], sm_col, deg_parts[0], deg_parts[1], W1)

    def mid_layer(y_in, b_in, w_next):
        parts = _sc_edge_pass(y_in, src, dst, ones_msk, pad_idx)
        return pl.pallas_call(
            _tc_mid_body,
            grid=(NBLK,),
            in_specs=[_row_spec(H), _row_spec(H), _row_spec(H), _row_spec(1),
                      _full_spec(1, H), _full_spec(H, H)],
            out_specs=_row_spec(H),
            out_shape=jax.ShapeDtypeStruct((NP, H), f32),
        )(parts[0], parts[1], y_in, dis, b_in[None, :], w_next)

    y2 = mid_layer(y1, b1, W2)
    y3 = mid_layer(y2, b2, W3)

    parts3 = _sc_edge_pass(y3, src, dst, ones_msk, pad_idx)
    out = pl.pallas_call(
        _tc4_body,
        grid=(NBLK,),
        in_specs=[_row_spec(H), _row_spec(H), _row_spec(H), _row_spec(1),
                  _full_spec(1, H), _row_spec(1), _full_spec(H, C),
                  _full_spec(1, C)],
        out_specs=_full_spec(G, C),
        out_shape=jax.ShapeDtypeStruct((G, C), f32),
        scratch_shapes=[pltpu.VMEM((G, H), f32), pltpu.VMEM((G, 1), f32)],
    )(parts3[0], parts3[1], y3, dis, b3[None, :], bat_col, Wl, bl[None, :])
    return out


# speedup vs baseline: 13.3123x; 1.0641x over previous
import functools

import jax
import jax.numpy as jnp
from jax import lax
from jax.experimental import pallas as pl
from jax.experimental.pallas import tpu as pltpu
from jax.experimental.pallas import tpu_sc as plsc

N = 10000
E = 320000
D = 128
H = 128
C = 16
G = 64

NC = 2
NS = 16
NW = NC * NS
EPW = E // NW
K = 80
NB = EPW // K
NP = 10240
ZROW = N
ZB = 32
RPT = NP // NS
CB = 25
NCH = NB // CB

_mesh = plsc.VectorSubcoreMesh(core_axis_name="c", subcore_axis_name="s")


def _zero_vmem(ref, nrow, ncol):
    def body(i, carry):
        for c in range(ncol // 16):
            ref[i, pl.ds(c * 16, 16)] = jnp.zeros((16,), jnp.float32)
        return carry
    lax.fori_loop(0, nrow, body, 0)


@functools.partial(
    pl.kernel,
    out_type=jax.ShapeDtypeStruct((NC, NP, D), jnp.float32),
    mesh=_mesh,
    scratch_types=[
        pltpu.VMEM((CB, K), jnp.int32),
        pltpu.VMEM((CB, K), jnp.int32),
        pltpu.VMEM((CB, K), jnp.float32),
        pltpu.VMEM((K,), jnp.int32),
        pltpu.VMEM((K,), jnp.int32),
        pltpu.VMEM((K, D), jnp.float32),
        pltpu.VMEM((K, D), jnp.float32),
        pltpu.VMEM((ZB, D), jnp.float32),
        pltpu.VMEM_SHARED((NP, D), jnp.float32),
        pltpu.SemaphoreType.DMA,
        pltpu.SemaphoreType.DMA,
    ],
)
def _sc_edge_pass(x_hbm, src_hbm, dst_hbm, msk_hbm, pad_hbm, out_hbm,
                  src_v, dst_v, msk_v, didx_v, pad_v, rows0_v,
                  rows1_v, zf_v, acc_sh, sem0, sem1):
    cid = lax.axis_index("c")
    sid = lax.axis_index("s")
    wid = cid * NS + sid

    _zero_vmem(zf_v, ZB, D)

    base = sid * RPT
    for t in range(RPT // ZB):
        pltpu.sync_copy(zf_v, acc_sh.at[pl.ds(base + t * ZB, ZB)])
    plsc.subcore_barrier()

    pltpu.sync_copy(pad_hbm, pad_v)

    def mk_dst(jj):
        for i in range(K // 16):
            m = msk_v[jj, pl.ds(i * 16, 16)]
            d = dst_v[jj, pl.ds(i * 16, 16)]
            t = pad_v[pl.ds(i * 16, 16)]
            didx_v[pl.ds(i * 16, 16)] = jnp.where(m > 0.5, d, t)

    for c in range(NCH):
        pltpu.sync_copy(src_hbm.at[wid, c], src_v)
        pltpu.sync_copy(dst_hbm.at[wid, c], dst_v)
        pltpu.sync_copy(msk_hbm.at[wid, c], msk_v)

        pltpu.async_copy(x_hbm.at[src_v.at[0]], rows0_v, sem0)

        def pair(g, carry):
            j = 2 * g
            pltpu.async_copy(x_hbm.at[src_v.at[j + 1]], rows1_v, sem1)
            pltpu.make_async_copy(x_hbm.at[src_v.at[j]], rows0_v, sem0).wait()
            mk_dst(j)
            pltpu.sync_copy(rows0_v, acc_sh.at[didx_v], add=True)

            @pl.when(j + 2 < CB)
            def _next():
                pltpu.async_copy(x_hbm.at[src_v.at[j + 2]], rows0_v, sem0)

            pltpu.make_async_copy(x_hbm.at[src_v.at[j + 1]], rows1_v, sem1).wait()
            mk_dst(j + 1)
            pltpu.sync_copy(rows1_v, acc_sh.at[didx_v], add=True)
            return carry

        lax.fori_loop(0, CB // 2, pair, 0)
        pltpu.make_async_copy(x_hbm.at[src_v.at[CB - 1]], rows0_v, sem0).wait()
        mk_dst(CB - 1)
        pltpu.sync_copy(rows0_v, acc_sh.at[didx_v], add=True)

    plsc.subcore_barrier()
    pltpu.sync_copy(acc_sh.at[pl.ds(base, RPT)], out_hbm.at[cid, pl.ds(base, RPT)])


BR = 128
NBLK = NP // BR


def _dis_from_deg(d0_blk, d1_blk):
    deg = d0_blk[:, :1] + d1_blk[:, :1] + 1.0
    return lax.rsqrt(jnp.maximum(deg, 1e-12))


def _tc1_body(x_ref, a0_ref, a1_ref, sm_ref, d0_ref, d1_ref, w_ref,
              y_ref, dis_ref):
    sm = sm_ref[...]
    x2 = a0_ref[...] + a1_ref[...]
    xm = sm * x2 + (1.0 - sm) * x_ref[...]
    dis = _dis_from_deg(d0_ref[...], d1_ref[...])
    y_ref[...] = dis * jnp.dot(xm, w_ref[...], preferred_element_type=jnp.float32)
    dis_ref[...] = dis


def _tc_mid_body(a0_ref, a1_ref, y_ref, dis_ref, b_ref, w_ref, out_ref):
    dis = dis_ref[...]
    h = dis * (a0_ref[...] + a1_ref[...] + y_ref[...]) + b_ref[...]
    h = jnp.maximum(h, 0.0)
    out_ref[...] = dis * jnp.dot(h, w_ref[...], preferred_element_type=jnp.float32)


def _tc4_body(a0_ref, a1_ref, y_ref, dis_ref, b_ref, bat_ref, wl_ref, bl_ref,
              out_ref, sums, cnt):
    i = pl.program_id(0)

    @pl.when(i == 0)
    def _init():
        sums[...] = jnp.zeros_like(sums)
        cnt[...] = jnp.zeros_like(cnt)

    dis = dis_ref[...]
    h = dis * (a0_ref[...] + a1_ref[...] + y_ref[...]) + b_ref[...]
    gids = lax.broadcasted_iota(jnp.int32, (BR, G), 1)
    oh = (bat_ref[...] == gids).astype(jnp.float32)
    sums[...] += lax.dot_general(oh, h, (((0,), (0,)), ((), ())),
                                 preferred_element_type=jnp.float32)
    cnt[...] += lax.dot_general(oh, jnp.ones((BR, 1), jnp.float32),
                                (((0,), (0,)), ((), ())),
                                preferred_element_type=jnp.float32)

    @pl.when(i == NBLK - 1)
    def _fin():
        pooled = sums[...] / jnp.maximum(cnt[...], 1.0)
        out_ref[...] = jnp.dot(pooled, wl_ref[...],
                               preferred_element_type=jnp.float32) + bl_ref[...]


def _row_spec(cols):
    return pl.BlockSpec((BR, cols), lambda i: (i, 0))


def _full_spec(rows, cols):
    return pl.BlockSpec((rows, cols), lambda i: (0, 0))


def kernel(x, edge_index, supernode_mask, edge_mask, batch,
           W1, b1, W2, b2, W3, b3, Wl, bl):
    f32 = jnp.float32
    src = edge_index[0].reshape(NW, NCH, CB, K)
    dst = edge_index[1].reshape(NW, NCH, CB, K)
    msk = edge_mask.reshape(NW, NCH, CB, K).astype(f32)

    pad = NP - N
    x_pad = jnp.concatenate([x, jnp.zeros((pad, D), f32)], axis=0)
    sm_col = jnp.concatenate(
        [supernode_mask.astype(f32), jnp.zeros((pad,), f32)])[:, None]
    bat_col = jnp.concatenate(
        [batch, jnp.full((pad,), G, jnp.int32)])[:, None]

    ones_msk = jnp.ones((NW, NCH, CB, K), f32)

    pad_idx = (N + jnp.arange(K, dtype=jnp.int32) % (NP - N))
    e0_table = jnp.zeros((NP, D), f32).at[:, 0].set(1.0)
    deg_parts = _sc_edge_pass(e0_table, dst, dst, ones_msk, pad_idx)
    parts_a = _sc_edge_pass(x_pad, src, dst, msk, pad_idx)

    y1, dis = pl.pallas_call(
        _tc1_body,
        grid=(NBLK,),
        in_specs=[_row_spec(D), _row_spec(D), _row_spec(D), _row_spec(1),
                  _row_spec(D), _row_spec(D), _full_spec(D, H)],
        out_specs=[_row_spec(H), _row_spec(1)],
        out_shape=[jax.ShapeDtypeStruct((NP, H), f32),
                   jax.ShapeDtypeStruct((NP, 1), f32)],
    )(x_pad, parts_a[0], parts_a[1], sm_col, deg_parts[0], deg_parts[1], W1)

    def mid_layer(y_in, b_in, w_next):
        parts = _sc_edge_pass(y_in, src, dst, ones_msk, pad_idx)
        return pl.pallas_call(
            _tc_mid_body,
            grid=(NBLK,),
            in_specs=[_row_spec(H), _row_spec(H), _row_spec(H), _row_spec(1),
                      _full_spec(1, H), _full_spec(H, H)],
            out_specs=_row_spec(H),
            out_shape=jax.ShapeDtypeStruct((NP, H), f32),
        )(parts[0], parts[1], y_in, dis, b_in[None, :], w_next)

    y2 = mid_layer(y1, b1, W2)
    y3 = mid_layer(y2, b2, W3)

    parts3 = _sc_edge_pass(y3, src, dst, ones_msk, pad_idx)
    out = pl.pallas_call(
        _tc4_body,
        grid=(NBLK,),
        in_specs=[_row_spec(H), _row_spec(H), _row_spec(H), _row_spec(1),
                  _full_spec(1, H), _row_spec(1), _full_spec(H, C),
                  _full_spec(1, C)],
        out_specs=_full_spec(G, C),
        out_shape=jax.ShapeDtypeStruct((G, C), f32),
        scratch_shapes=[pltpu.VMEM((G, H), f32), pltpu.VMEM((G, 1), f32)],
    )(parts3[0], parts3[1], y3, dis, b3[None, :], bat_col, Wl, bl[None, :])
    return out


# speedup vs baseline: 14.8574x; 1.1161x over previous
import functools

import jax
import jax.numpy as jnp
from jax import lax
from jax.experimental import pallas as pl
from jax.experimental.pallas import tpu as pltpu
from jax.experimental.pallas import tpu_sc as plsc

N = 10000
E = 320000
D = 128
H = 128
C = 16
G = 64

NC = 2
NS = 16
NW = NC * NS
EPW = E // NW
K = 80
NB = EPW // K
NP = 10240
ZROW = N
ZB = 32
RPT = NP // NS
CB = 25
NCH = NB // CB

_mesh = plsc.VectorSubcoreMesh(core_axis_name="c", subcore_axis_name="s")


def _zero_vmem(ref, nrow, ncol):
    def body(i, carry):
        for c in range(ncol // 16):
            ref[i, pl.ds(c * 16, 16)] = jnp.zeros((16,), jnp.float32)
        return carry
    lax.fori_loop(0, nrow, body, 0)


@functools.partial(
    pl.kernel,
    out_type=jax.ShapeDtypeStruct((NC, NP, D), jnp.float32),
    mesh=_mesh,
    scratch_types=[
        pltpu.VMEM((CB, K), jnp.int32),
        pltpu.VMEM((CB, K), jnp.int32),
        pltpu.VMEM((CB, K), jnp.float32),
        pltpu.VMEM((K,), jnp.int32),
        pltpu.VMEM((K,), jnp.int32),
        pltpu.VMEM((K, D), jnp.float32),
        pltpu.VMEM((K, D), jnp.float32),
        pltpu.VMEM((K, D), jnp.float32),
        pltpu.VMEM((ZB, D), jnp.float32),
        pltpu.VMEM_SHARED((NP, D), jnp.float32),
        pltpu.SemaphoreType.DMA,
        pltpu.SemaphoreType.DMA,
        pltpu.SemaphoreType.DMA,
    ],
)
def _sc_edge_pass(x_hbm, src_hbm, dst_hbm, msk_hbm, pad_hbm, out_hbm,
                  src_v, dst_v, msk_v, didx_v, pad_v, rows0_v,
                  rows1_v, rows2_v, zf_v, acc_sh, sem0, sem1, sem2):
    cid = lax.axis_index("c")
    sid = lax.axis_index("s")
    wid = cid * NS + sid

    _zero_vmem(zf_v, ZB, D)

    base = sid * RPT
    for t in range(RPT // ZB):
        pltpu.sync_copy(zf_v, acc_sh.at[pl.ds(base + t * ZB, ZB)])
    plsc.subcore_barrier()

    pltpu.sync_copy(pad_hbm, pad_v)

    def mk_dst(jj):
        for i in range(K // 16):
            m = msk_v[jj, pl.ds(i * 16, 16)]
            d = dst_v[jj, pl.ds(i * 16, 16)]
            t = pad_v[pl.ds(i * 16, 16)]
            didx_v[pl.ds(i * 16, 16)] = jnp.where(m > 0.5, d, t)

    for c in range(NCH):
        pltpu.sync_copy(src_hbm.at[wid, c], src_v)
        pltpu.sync_copy(dst_hbm.at[wid, c], dst_v)
        pltpu.sync_copy(msk_hbm.at[wid, c], msk_v)

        bufs = ((rows0_v, sem0), (rows1_v, sem1), (rows2_v, sem2))
        pltpu.async_copy(x_hbm.at[src_v.at[0]], rows0_v, sem0)
        pltpu.async_copy(x_hbm.at[src_v.at[1]], rows1_v, sem1)

        def triple(t, carry):
            j = 3 * t
            for b in range(3):
                rows_b, sem_b = bufs[(b + 2) % 3]
                jj = j + b + 2

                @pl.when(jj < CB)
                def _issue():
                    pltpu.async_copy(x_hbm.at[src_v.at[jj]], rows_b, sem_b)

                rows_w, sem_w = bufs[b]
                pltpu.make_async_copy(
                    x_hbm.at[src_v.at[j + b]], rows_w, sem_w).wait()
                mk_dst(j + b)
                pltpu.sync_copy(rows_w, acc_sh.at[didx_v], add=True)
            return carry

        lax.fori_loop(0, CB // 3, triple, 0)
        for j in range((CB // 3) * 3, CB):
            rows_w, sem_w = bufs[j % 3]
            pltpu.make_async_copy(
                x_hbm.at[src_v.at[j]], rows_w, sem_w).wait()
            mk_dst(j)
            pltpu.sync_copy(rows_w, acc_sh.at[didx_v], add=True)

    plsc.subcore_barrier()
    pltpu.sync_copy(acc_sh.at[pl.ds(base, RPT)], out_hbm.at[cid, pl.ds(base, RPT)])


BR = 128
NBLK = NP // BR


def _dis_from_deg(d0_blk, d1_blk):
    deg = d0_blk[:, :1] + d1_blk[:, :1] + 1.0
    return lax.rsqrt(jnp.maximum(deg, 1e-12))


def _tc1_body(x_ref, a0_ref, a1_ref, sm_ref, d0_ref, d1_ref, w_ref,
              y_ref, dis_ref):
    sm = sm_ref[...]
    x2 = a0_ref[...] + a1_ref[...]
    xm = sm * x2 + (1.0 - sm) * x_ref[...]
    dis = _dis_from_deg(d0_ref[...], d1_ref[...])
    y_ref[...] = dis * jnp.dot(xm, w_ref[...], preferred_element_type=jnp.float32)
    dis_ref[...] = dis


def _tc_mid_body(a0_ref, a1_ref, y_ref, dis_ref, b_ref, w_ref, out_ref):
    dis = dis_ref[...]
    h = dis * (a0_ref[...] + a1_ref[...] + y_ref[...]) + b_ref[...]
    h = jnp.maximum(h, 0.0)
    out_ref[...] = dis * jnp.dot(h, w_ref[...], preferred_element_type=jnp.float32)


def _tc4_body(a0_ref, a1_ref, y_ref, dis_ref, b_ref, bat_ref, wl_ref, bl_ref,
              out_ref, sums, cnt):
    i = pl.program_id(0)

    @pl.when(i == 0)
    def _init():
        sums[...] = jnp.zeros_like(sums)
        cnt[...] = jnp.zeros_like(cnt)

    dis = dis_ref[...]
    h = dis * (a0_ref[...] + a1_ref[...] + y_ref[...]) + b_ref[...]
    gids = lax.broadcasted_iota(jnp.int32, (BR, G), 1)
    oh = (bat_ref[...] == gids).astype(jnp.float32)
    sums[...] += lax.dot_general(oh, h, (((0,), (0,)), ((), ())),
                                 preferred_element_type=jnp.float32)
    cnt[...] += lax.dot_general(oh, jnp.ones((BR, 1), jnp.float32),
                                (((0,), (0,)), ((), ())),
                                preferred_element_type=jnp.float32)

    @pl.when(i == NBLK - 1)
    def _fin():
        pooled = sums[...] / jnp.maximum(cnt[...], 1.0)
        out_ref[...] = jnp.dot(pooled, wl_ref[...],
                               preferred_element_type=jnp.float32) + bl_ref[...]


def _row_spec(cols):
    return pl.BlockSpec((BR, cols), lambda i: (i, 0))


def _full_spec(rows, cols):
    return pl.BlockSpec((rows, cols), lambda i: (0, 0))


def kernel(x, edge_index, supernode_mask, edge_mask, batch,
           W1, b1, W2, b2, W3, b3, Wl, bl):
    f32 = jnp.float32
    src = edge_index[0].reshape(NW, NCH, CB, K)
    dst = edge_index[1].reshape(NW, NCH, CB, K)
    msk = edge_mask.reshape(NW, NCH, CB, K).astype(f32)

    pad = NP - N
    x_pad = jnp.concatenate([x, jnp.zeros((pad, D), f32)], axis=0)
    sm_col = jnp.concatenate(
        [supernode_mask.astype(f32), jnp.zeros((pad,), f32)])[:, None]
    bat_col = jnp.concatenate(
        [batch, jnp.full((pad,), G, jnp.int32)])[:, None]

    ones_msk = jnp.ones((NW, NCH, CB, K), f32)

    pad_idx = (N + jnp.arange(K, dtype=jnp.int32) % (NP - N))
    e0_table = jnp.zeros((NP, D), f32).at[:, 0].set(1.0)
    deg_parts = _sc_edge_pass(e0_table, dst, dst, ones_msk, pad_idx)
    parts_a = _sc_edge_pass(x_pad, src, dst, msk, pad_idx)

    y1, dis = pl.pallas_call(
        _tc1_body,
        grid=(NBLK,),
        in_specs=[_row_spec(D), _row_spec(D), _row_spec(D), _row_spec(1),
                  _row_spec(D), _row_spec(D), _full_spec(D, H)],
        out_specs=[_row_spec(H), _row_spec(1)],
        out_shape=[jax.ShapeDtypeStruct((NP, H), f32),
                   jax.ShapeDtypeStruct((NP, 1), f32)],
    )(x_pad, parts_a[0], parts_a[1], sm_col, deg_parts[0], deg_parts[1], W1)

    def mid_layer(y_in, b_in, w_next):
        parts = _sc_edge_pass(y_in, src, dst, ones_msk, pad_idx)
        return pl.pallas_call(
            _tc_mid_body,
            grid=(NBLK,),
            in_specs=[_row_spec(H), _row_spec(H), _row_spec(H), _row_spec(1),
                      _full_spec(1, H), _full_spec(H, H)],
            out_specs=_row_spec(H),
            out_shape=jax.ShapeDtypeStruct((NP, H), f32),
        )(parts[0], parts[1], y_in, dis, b_in[None, :], w_next)

    y2 = mid_layer(y1, b1, W2)
    y3 = mid_layer(y2, b2, W3)

    parts3 = _sc_edge_pass(y3, src, dst, ones_msk, pad_idx)
    out = pl.pallas_call(
        _tc4_body,
        grid=(NBLK,),
        in_specs=[_row_spec(H), _row_spec(H), _row_spec(H), _row_spec(1),
                  _full_spec(1, H), _row_spec(1), _full_spec(H, C),
                  _full_spec(1, C)],
        out_specs=_full_spec(G, C),
        out_shape=jax.ShapeDtypeStruct((G, C), f32),
        scratch_shapes=[pltpu.VMEM((G, H), f32), pltpu.VMEM((G, 1), f32)],
    )(parts3[0], parts3[1], y3, dis, b3[None, :], bat_col, Wl, bl[None, :])
    return out
